# baseline (speedup 1.0000x reference)
import jax
import jax.numpy as jnp
from jax import lax
from jax.experimental import pallas as pl
from jax.experimental.pallas import tpu as pltpu
from jax.experimental.pallas import tpu_sc as plsc

N = 10000
E = 320000
D = 128
NC = 2
NS = 16
NW = NC * NS
L = 16

CH = 80
EPS = E // NW
EPW = 10080
BLKC = 21
NBLKW = 6
CHKR = 80
NZCHK = N // CHKR

_GDN = lax.GatherDimensionNumbers(
    offset_dims=(), collapsed_slice_dims=(0,), start_index_map=(0,))


def _sc_body(x_hbm, src_hbm, dst_hbm, ef_hbm, out0_hbm, out1_hbm,
             acc_s, src_v, dst_v, ef_v, r0, r1, r2,
             g0, g1, g2, s0, s1, s2, isem):
    c = lax.axis_index("c")
    s = lax.axis_index("s")
    rows = (r0, r1, r2)
    gsems = (g0, g1, g2)
    ssems = (s0, s1, s2)

    def _zero_row(i, carry):
        for k in range(D // L):
            r0[i, pl.ds(k * L, L)] = jnp.zeros((L,), jnp.float32)
        return carry
    lax.fori_loop(0, CHKR, _zero_row, None)
    for k in range(8):
        t = s + k * NS

        @pl.when(t < NZCHK)
        def _():
            pltpu.sync_copy(r0, acc_s.at[pl.ds(t * CHKR, CHKR)])
    plsc.subcore_barrier()

    w = c * NS + s
    base = w * NBLKW
    idx_views = ((src_hbm, src_v), (dst_hbm, dst_v), (ef_hbm, ef_v))

    def _scale(rv, row):
        for g16 in range(CH // L):
            efg = ef_v[row, pl.ds(g16 * L, L)]
            for jl in range(L):
                efb = lax.gather(
                    efg, jnp.full((L, 1), jl, jnp.int32), _GDN,
                    slice_sizes=(1,),
                    mode=lax.GatherScatterMode.PROMISE_IN_BOUNDS)
                jj = g16 * L + jl
                for k in range(D // L):
                    rv[jj, pl.ds(k * L, L)] = rv[jj, pl.ds(k * L, L)] * efb

    for a_hbm, a_v in idx_views:
        pltpu.sync_copy(a_hbm.at[base], a_v.at[pl.ds(0, BLKC)])
    for a_hbm, a_v in idx_views:
        pltpu.async_copy(a_hbm.at[base + 1], a_v.at[pl.ds(BLKC, BLKC)], isem)
    pltpu.async_copy(x_hbm.at[src_v.at[0]], r0, g0)
    pltpu.async_copy(x_hbm.at[src_v.at[1]], r1, g1)

    def _block(blk, carry):
        par = blk % 2
        parn = 1 - par

        @pl.when((blk > 0) & (blk < NBLKW - 1))
        def _():
            for a_hbm, a_v in idx_views:
                pltpu.async_copy(a_hbm.at[base + blk + 1],
                                 a_v.at[pl.ds(parn * BLKC, BLKC)], isem)

        def _triple(g, c2):
            @pl.when((g == 6) & (blk < NBLKW - 1))
            def _():
                for a_hbm, a_v in idx_views:
                    pltpu.make_async_copy(
                        a_hbm.at[base + blk + 1],
                        a_v.at[pl.ds(parn * BLKC, BLKC)], isem).wait()

            for b3 in range(3):
                j = g * 3 + b3
                row = par * BLKC + j
                rv = rows[b3]
                pltpu.make_async_copy(
                    x_hbm.at[src_v.at[row]], rv, gsems[b3]).wait()
                _scale(rv, row)
                pltpu.async_copy(rv, acc_s.at[dst_v.at[row]], ssems[b3],
                                 add=True)
                pb = (b3 + 2) % 3

                @pl.when(blk * BLKC + j > 0)
                def _():
                    pltpu.make_async_copy(
                        rows[pb], acc_s.at[dst_v.at[row]], ssems[pb]).wait()

                jla = j + 2
                cross = jla >= BLKC
                rowla = jnp.where(cross, parn * BLKC + jla - BLKC,
                                  par * BLKC + jla)

                @pl.when(jnp.logical_or(jnp.logical_not(cross),
                                        blk < NBLKW - 1))
                def _():
                    pltpu.async_copy(x_hbm.at[src_v.at[rowla]], rows[pb],
                                     gsems[pb])
            return c2
        lax.fori_loop(0, BLKC // 3, _triple, None)
        return carry
    lax.fori_loop(0, NBLKW, _block, None)

    pltpu.make_async_copy(
        rows[2], acc_s.at[dst_v.at[2 * BLKC - 1]], ssems[2]).wait()
    plsc.subcore_barrier()

    for k in range(8):
        t = s + k * NS

        @pl.when(t < NZCHK)
        def _():
            row = t * CHKR

            @pl.when(c == 0)
            def _():
                pltpu.sync_copy(acc_s.at[pl.ds(row, CHKR)],
                                out0_hbm.at[pl.ds(row, CHKR)])

            @pl.when(c == 1)
            def _():
                pltpu.sync_copy(acc_s.at[pl.ds(row, CHKR)],
                                out1_hbm.at[pl.ds(row, CHKR)])


@jax.jit
def _sc_segment_sum(x, src4, dst4, ef4):
    mesh = plsc.VectorSubcoreMesh(core_axis_name="c", subcore_axis_name="s")
    return pl.kernel(
        _sc_body,
        out_type=(jax.ShapeDtypeStruct((N, D), jnp.float32),
                  jax.ShapeDtypeStruct((N, D), jnp.float32)),
        mesh=mesh,
        scratch_types=[
            pltpu.VMEM_SHARED((N, D), jnp.float32),
            pltpu.VMEM((2 * BLKC, CH), jnp.int32),
            pltpu.VMEM((2 * BLKC, CH), jnp.int32),
            pltpu.VMEM((2 * BLKC, CH), jnp.float32),
            pltpu.VMEM((CH, D), jnp.float32),
            pltpu.VMEM((CH, D), jnp.float32),
            pltpu.VMEM((CH, D), jnp.float32),
            pltpu.SemaphoreType.DMA,
            pltpu.SemaphoreType.DMA,
            pltpu.SemaphoreType.DMA,
            pltpu.SemaphoreType.DMA,
            pltpu.SemaphoreType.DMA,
            pltpu.SemaphoreType.DMA,
            pltpu.SemaphoreType.DMA,
        ],
    )(x, src4, dst4, ef4)


MT = 2000


def _mm_body(x_ref, r0_ref, r1_ref, w_ref, b_ref, o_ref):
    dn = (((1,), (1,)), ((), ()))
    o_ref[...] = (
        lax.dot_general(x_ref[...], w_ref[:, :D], dn,
                        preferred_element_type=jnp.float32)
        + lax.dot_general(r0_ref[...] + r1_ref[...], w_ref[:, D:], dn,
                          preferred_element_type=jnp.float32)
        + b_ref[...])


@jax.jit
def _tc_linear(x, r0, r1, W, b2):
    return pl.pallas_call(
        _mm_body,
        grid=(N // MT,),
        in_specs=[
            pl.BlockSpec((MT, D), lambda i: (i, 0)),
            pl.BlockSpec((MT, D), lambda i: (i, 0)),
            pl.BlockSpec((MT, D), lambda i: (i, 0)),
            pl.BlockSpec((D, 2 * D), lambda i: (0, 0)),
            pl.BlockSpec((1, D), lambda i: (0, 0)),
        ],
        out_specs=pl.BlockSpec((MT, D), lambda i: (i, 0)),
        out_shape=jax.ShapeDtypeStruct((N, D), jnp.float32),
    )(x, r0, r1, W, b2)


def _pad_split(a):
    a = a.reshape(NW, EPS)
    a = jnp.pad(a, ((0, 0), (0, EPW - EPS)))
    return a.reshape(NW * NBLKW, BLKC, CH)


def kernel(node_feats, edge_index, edge_feats, W, b):
    src4 = _pad_split(edge_index[0])
    dst4 = _pad_split(edge_index[1])
    ef4 = _pad_split(edge_feats.reshape(E))
    r0, r1 = _sc_segment_sum(node_feats, src4, dst4, ef4)
    return _tc_linear(node_feats, r0, r1, W, b.reshape(1, D))


# speedup vs baseline: 1.3263x; 1.3263x over previous
import jax
import jax.numpy as jnp
from jax import lax
from jax.experimental import pallas as pl
from jax.experimental.pallas import tpu as pltpu
from jax.experimental.pallas import tpu_sc as plsc

N = 10000
E = 320000
D = 128
NC = 2
NS = 16
NW = NC * NS
L = 16

CH = 80
BLK = 25
NBLK = 5
EBLOCKS = NW * NBLK
CHKR = 80
NZCHK = N // CHKR

_GDN = lax.GatherDimensionNumbers(
    offset_dims=(), collapsed_slice_dims=(0,), start_index_map=(0,))


def _sc_body(x_hbm, src_hbm, dst_hbm, ef_hbm, out0_hbm, out1_hbm,
             acc_s, src_v, dst_v, ef_v, rows_a, rows_b,
             gsem_a, gsem_b, ssem_a, ssem_b):
    c = lax.axis_index("c")
    s = lax.axis_index("s")

    def _zero_row(i, carry):
        for k in range(D // L):
            rows_a[i, pl.ds(k * L, L)] = jnp.zeros((L,), jnp.float32)
        return carry
    lax.fori_loop(0, CHKR, _zero_row, None)
    for k in range(8):
        t = s + k * NS

        @pl.when(t < NZCHK)
        def _():
            pltpu.sync_copy(rows_a, acc_s.at[pl.ds(t * CHKR, CHKR)])
    plsc.subcore_barrier()

    w = c * NS + s

    def _scale(rows_v, j):
        for g in range(CH // L):
            efg = ef_v[j, pl.ds(g * L, L)]
            for jl in range(L):
                efb = lax.gather(
                    efg, jnp.full((L, 1), jl, jnp.int32), _GDN,
                    slice_sizes=(1,),
                    mode=lax.GatherScatterMode.PROMISE_IN_BOUNDS)
                jj = g * L + jl
                for k in range(D // L):
                    rows_v[jj, pl.ds(k * L, L)] = (
                        rows_v[jj, pl.ds(k * L, L)] * efb)

    def _block(ob, carry):
        blk = w * NBLK + ob
        pltpu.sync_copy(src_hbm.at[blk], src_v)
        pltpu.sync_copy(dst_hbm.at[blk], dst_v)
        pltpu.sync_copy(ef_hbm.at[blk], ef_v)

        @pl.when(ob > 0)
        def _():
            pltpu.make_async_copy(
                rows_a, acc_s.at[dst_v.at[0]], ssem_a).wait()
        pltpu.async_copy(x_hbm.at[src_v.at[0]], rows_a, gsem_a)

        def _pair(p, carry2):
            j = 2 * p
            pltpu.make_async_copy(x_hbm.at[src_v.at[j]], rows_a, gsem_a).wait()

            @pl.when(p > 0)
            def _():
                pltpu.make_async_copy(
                    rows_b, acc_s.at[dst_v.at[j]], ssem_b).wait()
            pltpu.async_copy(x_hbm.at[src_v.at[j + 1]], rows_b, gsem_b)
            _scale(rows_a, j)
            pltpu.async_copy(rows_a, acc_s.at[dst_v.at[j]], ssem_a, add=True)

            pltpu.make_async_copy(
                x_hbm.at[src_v.at[j + 1]], rows_b, gsem_b).wait()
            pltpu.make_async_copy(
                rows_a, acc_s.at[dst_v.at[j]], ssem_a).wait()
            pltpu.async_copy(x_hbm.at[src_v.at[j + 2]], rows_a, gsem_a)
            _scale(rows_b, j + 1)
            pltpu.async_copy(rows_b, acc_s.at[dst_v.at[j + 1]], ssem_b,
                             add=True)
            return carry2
        lax.fori_loop(0, BLK // 2, _pair, None)

        pltpu.make_async_copy(
            x_hbm.at[src_v.at[BLK - 1]], rows_a, gsem_a).wait()
        pltpu.make_async_copy(rows_b, acc_s.at[dst_v.at[0]], ssem_b).wait()
        _scale(rows_a, BLK - 1)
        pltpu.async_copy(rows_a, acc_s.at[dst_v.at[BLK - 1]], ssem_a,
                         add=True)
        return carry
    lax.fori_loop(0, NBLK, _block, None)
    pltpu.make_async_copy(rows_a, acc_s.at[dst_v.at[0]], ssem_a).wait()
    plsc.subcore_barrier()

    for k in range(8):
        t = s + k * NS

        @pl.when(t < NZCHK)
        def _():
            row = t * CHKR

            @pl.when(c == 0)
            def _():
                pltpu.sync_copy(acc_s.at[pl.ds(row, CHKR)],
                                out0_hbm.at[pl.ds(row, CHKR)])

            @pl.when(c == 1)
            def _():
                pltpu.sync_copy(acc_s.at[pl.ds(row, CHKR)],
                                out1_hbm.at[pl.ds(row, CHKR)])


@jax.jit
def _sc_segment_sum(x, src3, dst3, ef3):
    mesh = plsc.VectorSubcoreMesh(core_axis_name="c", subcore_axis_name="s")
    return pl.kernel(
        _sc_body,
        out_type=(jax.ShapeDtypeStruct((N, D), jnp.float32),
                  jax.ShapeDtypeStruct((N, D), jnp.float32)),
        mesh=mesh,
        scratch_types=[
            pltpu.VMEM_SHARED((N, D), jnp.float32),
            pltpu.VMEM((BLK, CH), jnp.int32),
            pltpu.VMEM((BLK, CH), jnp.int32),
            pltpu.VMEM((BLK, CH), jnp.float32),
            pltpu.VMEM((CH, D), jnp.float32),
            pltpu.VMEM((CH, D), jnp.float32),
            pltpu.SemaphoreType.DMA,
            pltpu.SemaphoreType.DMA,
            pltpu.SemaphoreType.DMA,
            pltpu.SemaphoreType.DMA,
        ],
    )(x, src3, dst3, ef3)


MT = 2000


def _mm_body(x_ref, r0_ref, r1_ref, w_ref, b_ref, o_ref):
    dn = (((1,), (1,)), ((), ()))
    o_ref[...] = (
        lax.dot_general(x_ref[...], w_ref[:, :D], dn,
                        preferred_element_type=jnp.float32)
        + lax.dot_general(r0_ref[...] + r1_ref[...], w_ref[:, D:], dn,
                          preferred_element_type=jnp.float32)
        + b_ref[...])


@jax.jit
def _tc_linear(x, r0, r1, W, b2):
    return pl.pallas_call(
        _mm_body,
        grid=(N // MT,),
        in_specs=[
            pl.BlockSpec((MT, D), lambda i: (i, 0)),
            pl.BlockSpec((MT, D), lambda i: (i, 0)),
            pl.BlockSpec((MT, D), lambda i: (i, 0)),
            pl.BlockSpec((D, 2 * D), lambda i: (0, 0)),
            pl.BlockSpec((1, D), lambda i: (0, 0)),
        ],
        out_specs=pl.BlockSpec((MT, D), lambda i: (i, 0)),
        out_shape=jax.ShapeDtypeStruct((N, D), jnp.float32),
    )(x, r0, r1, W, b2)


def kernel(node_feats, edge_index, edge_feats, W, b):
    src3 = edge_index[0].reshape(EBLOCKS, BLK, CH)
    dst3 = edge_index[1].reshape(EBLOCKS, BLK, CH)
    ef3 = edge_feats.reshape(EBLOCKS, BLK, CH)
    r0, r1 = _sc_segment_sum(node_feats, src3, dst3, ef3)
    return _tc_linear(node_feats, r0, r1, W, b.reshape(1, D))
